# baseline (speedup 1.0000x reference)
import functools

import jax
import jax.numpy as jnp
from jax import lax
from jax.experimental import pallas as pl
from jax.experimental.pallas import tpu as pltpu
from jax.experimental.pallas import tpu_sc as plsc

M = 1000000
B = 16384
D = 32
H1 = 64
H2 = 32

RB = 4096
G = 62
S = RB * G

NC = 2
NS = 16
NW = NC * NS
BPW = B // NW
CH = 128
NCH = BPW // CH

BLK = 2048


def _relayout_body(in0, in1, in2, in3, out_ref):
  cat = jnp.concatenate(
      [in0[...], in1[...], in2[...], in3[...]], axis=0)
  out_ref[...] = cat.T.astype(jnp.bfloat16)


def _tc_relayout(tt):
  return pl.pallas_call(
      _relayout_body,
      grid=(G,),
      in_specs=[pl.BlockSpec(
          (32, RB),
          lambda g, k=k: (0, jnp.minimum((k * S) // RB + g, M // RB)))
                for k in range(4)],
      out_specs=pl.BlockSpec((RB, 128), lambda g: (g, 0)),
      out_shape=jax.ShapeDtypeStruct((S, 128), jnp.bfloat16),
  )(tt, tt, tt, tt)


def _sc_gather(p2d, lin):
  mesh = plsc.VectorSubcoreMesh(core_axis_name="c", subcore_axis_name="s")

  @functools.partial(
      pl.kernel,
      out_type=jax.ShapeDtypeStruct((B, 128), jnp.bfloat16),
      mesh=mesh,
      compiler_params=pltpu.CompilerParams(use_tc_tiling_on_sc=False),
      scratch_types=[
          pltpu.VMEM((NCH, CH), jnp.int32),
          pltpu.VMEM((BPW, 128), jnp.bfloat16),
          pltpu.SemaphoreType.DMA,
      ],
  )
  def gather_kernel(p_hbm, lin_hbm, x_hbm, idx_v, rows_v, sem):
    wid = lax.axis_index("s") * NC + lax.axis_index("c")
    base = wid * BPW
    pltpu.sync_copy(p_hbm.at[pl.ds(wid * NCH, NCH)], idx_v)
    cps = [
        pltpu.async_copy(lin_hbm.at[idx_v.at[j]],
                         rows_v.at[pl.ds(j * CH, CH)], sem)
        for j in range(NCH)
    ]
    for c in cps:
      c.wait()
    pltpu.sync_copy(rows_v, x_hbm.at[pl.ds(base, BPW)])

  return gather_kernel(p2d, lin)


def _mlp_body(xu_ref, xb_ref, offu_ref, offb_ref, w1u_ref, w1b_ref, b1_ref,
              w2t_ref, b2_ref, o_ref):
  lane = lax.broadcasted_iota(jnp.int32, (BLK, 128), 1)
  offu = offu_ref[...]
  offb = offb_ref[...]
  zero = jnp.zeros((), jnp.bfloat16)
  xu = jnp.where((lane >= offu) & (lane < offu + D), xu_ref[...], zero)
  xb = jnp.where((lane >= offb) & (lane < offb + D), xb_ref[...], zero)
  h = lax.dot_general(xu, w1u_ref[...], (((1,), (0,)), ((), ())),
                      preferred_element_type=jnp.float32)
  h = h + lax.dot_general(xb, w1b_ref[...], (((1,), (0,)), ((), ())),
                          preferred_element_type=jnp.float32)
  h = jnp.maximum(h + b1_ref[...], 0.0).astype(jnp.bfloat16)
  o = lax.dot_general(h, w2t_ref[...], (((1,), (0,)), ((), ())),
                      preferred_element_type=jnp.float32)
  o_ref[...] = jnp.maximum(o + b2_ref[...], 0.0)


def _tc_mlp(xu, xb, offu, offb, w1u4t, w1b4t, b1r, w2t, b2r):
  grid = (B // BLK,)
  return pl.pallas_call(
      _mlp_body,
      grid=grid,
      in_specs=[
          pl.BlockSpec((BLK, 128), lambda i: (i, 0)),
          pl.BlockSpec((BLK, 128), lambda i: (i, 0)),
          pl.BlockSpec((BLK, 1), lambda i: (i, 0)),
          pl.BlockSpec((BLK, 1), lambda i: (i, 0)),
          pl.BlockSpec((128, H1), lambda i: (0, 0)),
          pl.BlockSpec((128, H1), lambda i: (0, 0)),
          pl.BlockSpec((1, H1), lambda i: (0, 0)),
          pl.BlockSpec((H1, H2), lambda i: (0, 0)),
          pl.BlockSpec((1, H2), lambda i: (0, 0)),
      ],
      out_specs=pl.BlockSpec((BLK, H2), lambda i: (i, 0)),
      out_shape=jax.ShapeDtypeStruct((B, H2), jnp.float32),
  )(xu, xb, offu, offb, w1u4t, w1b4t, b1r, w2t, b2r)


def _split(idx):
  idx = idx.astype(jnp.int32)
  k = ((idx >= S).astype(jnp.int32) + (idx >= 2 * S).astype(jnp.int32)
       + (idx >= 3 * S).astype(jnp.int32))
  p = idx - k * S
  return p.reshape(NW * NCH, CH), (k * D).reshape(B, 1)


def kernel(user_id, book_id, user_table, book_table, W1, b1, W2, b2):
  pu2d, offu = _split(user_id)
  pb2d, offb = _split(book_id)
  lin_u = _tc_relayout(user_table.T)
  xu = _sc_gather(pu2d, lin_u)
  lin_b = _tc_relayout(book_table.T)
  xb = _sc_gather(pb2d, lin_b)
  bf = jnp.bfloat16
  w1u4t = jnp.tile(W1[:, :D], (1, 4)).T.astype(bf)
  w1b4t = jnp.tile(W1[:, D:], (1, 4)).T.astype(bf)
  return _tc_mlp(xu, xb, offu, offb, w1u4t, w1b4t, b1.reshape(1, H1),
                 W2.T.astype(bf), b2.reshape(1, H2))


# speedup vs baseline: 2.5594x; 2.5594x over previous
import functools

import jax
import jax.numpy as jnp
from jax import lax
from jax.experimental import pallas as pl
from jax.experimental.pallas import tpu as pltpu
from jax.experimental.pallas import tpu_sc as plsc

M = 1000000
B = 16384
D = 32
H1 = 64
H2 = 32

RB = 4096
G = 62
S = RB * G

NC = 2
NS = 16
NW = NC * NS
BPW = B // NW
CH = 128
NCH = BPW // CH

BLK = 2048


def _relayout_body(in0, in1, in2, in3, out_ref):
  cat = jnp.concatenate(
      [in0[...], in1[...], in2[...], in3[...]], axis=0)
  out_ref[...] = cat.T


def _tc_relayout(tt):
  return pl.pallas_call(
      _relayout_body,
      grid=(G,),
      in_specs=[pl.BlockSpec(
          (32, RB),
          lambda g, k=k: (0, jnp.minimum((k * S) // RB + g, M // RB)))
                for k in range(4)],
      out_specs=pl.BlockSpec((RB, 128), lambda g: (g, 0)),
      out_shape=jax.ShapeDtypeStruct((S, 128), jnp.float32),
  )(tt, tt, tt, tt)


def _sc_gather(p2d, lin):
  mesh = plsc.VectorSubcoreMesh(core_axis_name="c", subcore_axis_name="s")

  @functools.partial(
      pl.kernel,
      out_type=jax.ShapeDtypeStruct((B, 128), jnp.float32),
      mesh=mesh,
      compiler_params=pltpu.CompilerParams(use_tc_tiling_on_sc=False),
      scratch_types=[
          pltpu.VMEM((NCH, CH), jnp.int32),
          pltpu.VMEM((BPW, 128), jnp.float32),
          pltpu.SemaphoreType.DMA,
      ],
  )
  def gather_kernel(p_hbm, lin_hbm, x_hbm, idx_v, rows_v, sem):
    wid = lax.axis_index("s") * NC + lax.axis_index("c")
    base = wid * BPW
    pltpu.sync_copy(p_hbm.at[pl.ds(wid * NCH, NCH)], idx_v)
    cps = [
        pltpu.async_copy(lin_hbm.at[idx_v.at[j]],
                         rows_v.at[pl.ds(j * CH, CH)], sem)
        for j in range(NCH)
    ]
    for c in cps:
      c.wait()
    pltpu.sync_copy(rows_v, x_hbm.at[pl.ds(base, BPW)])

  return gather_kernel(p2d, lin)


def _mlp_body(xu_ref, xb_ref, offu_ref, offb_ref, w1u_ref, w1b_ref, b1_ref,
              w2t_ref, b2_ref, o_ref):
  lane = lax.broadcasted_iota(jnp.int32, (BLK, 128), 1)
  offu = offu_ref[...]
  offb = offb_ref[...]
  xu = jnp.where((lane >= offu) & (lane < offu + D), xu_ref[...], 0.0)
  xb = jnp.where((lane >= offb) & (lane < offb + D), xb_ref[...], 0.0)
  h = lax.dot_general(xu, w1u_ref[...], (((1,), (0,)), ((), ())),
                      preferred_element_type=jnp.float32)
  h = h + lax.dot_general(xb, w1b_ref[...], (((1,), (0,)), ((), ())),
                          preferred_element_type=jnp.float32)
  h = jnp.maximum(h + b1_ref[...], 0.0)
  o = lax.dot_general(h, w2t_ref[...], (((1,), (0,)), ((), ())),
                      preferred_element_type=jnp.float32)
  o_ref[...] = jnp.maximum(o + b2_ref[...], 0.0)


def _tc_mlp(xu, xb, offu, offb, w1u4t, w1b4t, b1r, w2t, b2r):
  grid = (B // BLK,)
  return pl.pallas_call(
      _mlp_body,
      grid=grid,
      in_specs=[
          pl.BlockSpec((BLK, 128), lambda i: (i, 0)),
          pl.BlockSpec((BLK, 128), lambda i: (i, 0)),
          pl.BlockSpec((BLK, 1), lambda i: (i, 0)),
          pl.BlockSpec((BLK, 1), lambda i: (i, 0)),
          pl.BlockSpec((128, H1), lambda i: (0, 0)),
          pl.BlockSpec((128, H1), lambda i: (0, 0)),
          pl.BlockSpec((1, H1), lambda i: (0, 0)),
          pl.BlockSpec((H1, H2), lambda i: (0, 0)),
          pl.BlockSpec((1, H2), lambda i: (0, 0)),
      ],
      out_specs=pl.BlockSpec((BLK, H2), lambda i: (i, 0)),
      out_shape=jax.ShapeDtypeStruct((B, H2), jnp.float32),
  )(xu, xb, offu, offb, w1u4t, w1b4t, b1r, w2t, b2r)


def _split(idx):
  idx = idx.astype(jnp.int32)
  k = ((idx >= S).astype(jnp.int32) + (idx >= 2 * S).astype(jnp.int32)
       + (idx >= 3 * S).astype(jnp.int32))
  p = idx - k * S
  return p.reshape(NW * NCH, CH), (k * D).reshape(B, 1)


def kernel(user_id, book_id, user_table, book_table, W1, b1, W2, b2):
  pu2d, offu = _split(user_id)
  pb2d, offb = _split(book_id)
  lin_u = _tc_relayout(user_table.T)
  xu = _sc_gather(pu2d, lin_u)
  lin_b = _tc_relayout(book_table.T)
  xb = _sc_gather(pb2d, lin_b)
  w1u4t = jnp.tile(W1[:, :D], (1, 4)).T
  w1b4t = jnp.tile(W1[:, D:], (1, 4)).T
  return _tc_mlp(xu, xb, offu, offb, w1u4t, w1b4t, b1.reshape(1, H1),
                 W2.T, b2.reshape(1, H2))
